# baseline (speedup 1.0000x reference)
import functools

import jax
import jax.numpy as jnp
from jax import lax
from jax.experimental import pallas as pl
from jax.experimental.pallas import tpu as pltpu
from jax.experimental.pallas import tpu_sc as plsc

_TILE = 1024
_CHUNK = 128


def _vq_step(z_ref, z2_ref, m_ref, cb_ref, c2_ref, kiota_ref,
             idx_ref, sumsq_ref, cnt_ref):
    i = pl.program_id(0)
    z = z_ref[0]
    k = cb_ref.shape[0]

    dots = jax.lax.dot_general(
        cb_ref[...], z, (((1,), (1,)), ((), ())),
        preferred_element_type=jnp.float32)
    dist = (z2_ref[0] - 2.0 * dots) + c2_ref[...]

    minv = jnp.min(dist, axis=0, keepdims=True)
    kiota = jnp.broadcast_to(kiota_ref[...], dist.shape)
    idx = jnp.min(jnp.where(dist == minv, kiota, k),
                  axis=0, keepdims=True)
    idx_ref[0] = idx

    m = m_ref[0]
    psum = jnp.sum(minv * m, keepdims=True)
    pcnt = jnp.sum(m, keepdims=True)

    @pl.when(i == 0)
    def _init():
        sumsq_ref[...] = jnp.zeros((1, 1), jnp.float32)
        cnt_ref[...] = jnp.zeros((1, 1), jnp.float32)

    sumsq_ref[...] += psum
    cnt_ref[...] += pcnt


def _tc_indices_loss(z, mask, codebook):
    b, t, d = z.shape
    k = codebook.shape[0]
    n = b * t
    nt = n // _TILE

    z3 = z.reshape(nt, _TILE, d)
    z2 = jnp.sum(z * z, axis=-1).reshape(nt, 1, _TILE)
    m3 = mask.astype(jnp.float32).reshape(nt, 1, _TILE)
    c2 = jnp.sum(codebook * codebook, axis=-1).reshape(k, 1)
    kiota = jax.lax.broadcasted_iota(jnp.int32, (k, 1), 0)

    idx3, sumsq, cnt = pl.pallas_call(
        _vq_step,
        grid=(nt,),
        in_specs=[
            pl.BlockSpec((1, _TILE, d), lambda i: (i, 0, 0)),
            pl.BlockSpec((1, 1, _TILE), lambda i: (i, 0, 0)),
            pl.BlockSpec((1, 1, _TILE), lambda i: (i, 0, 0)),
            pl.BlockSpec((k, d), lambda i: (0, 0)),
            pl.BlockSpec((k, 1), lambda i: (0, 0)),
            pl.BlockSpec((k, 1), lambda i: (0, 0)),
        ],
        out_specs=[
            pl.BlockSpec((1, 1, _TILE), lambda i: (i, 0, 0)),
            pl.BlockSpec((1, 1), lambda i: (0, 0)),
            pl.BlockSpec((1, 1), lambda i: (0, 0)),
        ],
        out_shape=[
            jax.ShapeDtypeStruct((nt, 1, _TILE), jnp.int32),
            jax.ShapeDtypeStruct((1, 1), jnp.float32),
            jax.ShapeDtypeStruct((1, 1), jnp.float32),
        ],
        compiler_params=pltpu.CompilerParams(
            dimension_semantics=("arbitrary",),
        ),
    )(z3, z2, m3, codebook, c2, kiota)
    return idx3.reshape(b, t), sumsq[0, 0], cnt[0, 0]


def _sc_gather(codebook, indices):
    nrows, d = codebook.shape[0], codebook.shape[1]
    nidx = indices.shape[0]
    info = plsc.get_sparse_core_info()
    nw = info.num_cores * info.num_subcores
    b_per_w = nidx // nw
    nchunks = b_per_w // _CHUNK
    mesh = plsc.VectorSubcoreMesh(core_axis_name="c", subcore_axis_name="s")

    @functools.partial(
        pl.kernel, mesh=mesh,
        out_type=jax.ShapeDtypeStruct((nidx, d), jnp.float32),
        scratch_types=[
            pltpu.VMEM((_CHUNK,), jnp.int32),
            pltpu.VMEM((_CHUNK,), jnp.int32),
            pltpu.VMEM((_CHUNK, d), jnp.float32),
            pltpu.VMEM((_CHUNK, d), jnp.float32),
            pltpu.SemaphoreType.DMA,
            pltpu.SemaphoreType.DMA,
        ],
    )
    def gather_k(cb_hbm, idx_hbm, out_hbm,
                 idx0, idx1, rows0, rows1, sem0, sem1):
        wid = lax.axis_index("s") * info.num_cores + lax.axis_index("c")
        base = wid * b_per_w
        bufs = ((idx0, rows0, sem0), (idx1, rows1, sem1))
        pending = []
        for j in range(nchunks):
            iv, rv, sem = bufs[j % 2]
            off = base + j * _CHUNK
            pltpu.sync_copy(idx_hbm.at[pl.ds(off, _CHUNK)], iv)
            cp = pltpu.async_copy(cb_hbm.at[iv], rv, sem)
            pending.append((cp, rv, off))
            if len(pending) == 2:
                pcp, prv, poff = pending.pop(0)
                pcp.wait()
                pltpu.sync_copy(prv, out_hbm.at[pl.ds(poff, _CHUNK)])
        for pcp, prv, poff in pending:
            pcp.wait()
            pltpu.sync_copy(prv, out_hbm.at[pl.ds(poff, _CHUNK)])

    return gather_k(codebook, indices)


def kernel(z, mask, codebook):
    b, t, d = z.shape
    h = b // 2
    idx_a, sumsq_a, cnt_a = _tc_indices_loss(z[:h], mask[:h], codebook)
    idx_b, sumsq_b, cnt_b = _tc_indices_loss(z[h:], mask[h:], codebook)
    rows_a = _sc_gather(codebook, idx_a.reshape(h * t))
    rows_b = _sc_gather(codebook, idx_b.reshape(h * t))
    quantized = jnp.concatenate(
        [rows_a.reshape(h, t, d), rows_b.reshape(h, t, d)], axis=0)
    indices = jnp.concatenate([idx_a, idx_b], axis=0)
    denom = jnp.maximum(cnt_a + cnt_b, 1.0) * jnp.float32(d)
    sum_commit_loss = (sumsq_a + sumsq_b) / denom
    return quantized, indices, sum_commit_loss


# speedup vs baseline: 1.1878x; 1.1878x over previous
import functools

import jax
import jax.numpy as jnp
from jax import lax
from jax.experimental import pallas as pl
from jax.experimental.pallas import tpu as pltpu
from jax.experimental.pallas import tpu_sc as plsc

_TILE = 1024
_CHUNK = 128


def _vq_step(z_ref, z2_ref, m_ref, cb_ref, c2_ref, kiota_ref,
             idx_ref, sumsq_ref, cnt_ref):
    i = pl.program_id(0)
    z = z_ref[0]
    k = cb_ref.shape[0]

    dots = jax.lax.dot_general(
        cb_ref[...], z, (((1,), (1,)), ((), ())),
        preferred_element_type=jnp.float32)
    dist = (z2_ref[0] - 2.0 * dots) + c2_ref[...]

    minv = jnp.min(dist, axis=0, keepdims=True)
    kiota = jnp.broadcast_to(kiota_ref[...], dist.shape)
    idx = jnp.min(jnp.where(dist == minv, kiota, k),
                  axis=0, keepdims=True)
    idx_ref[0] = idx

    m = m_ref[0]
    psum = jnp.sum(minv * m, keepdims=True)
    pcnt = jnp.sum(m, keepdims=True)

    @pl.when(i == 0)
    def _init():
        sumsq_ref[...] = jnp.zeros((1, 1), jnp.float32)
        cnt_ref[...] = jnp.zeros((1, 1), jnp.float32)

    sumsq_ref[...] += psum
    cnt_ref[...] += pcnt


def _tc_indices_loss(z, mask, codebook):
    b, t, d = z.shape
    k = codebook.shape[0]
    n = b * t
    nt = n // _TILE

    z3 = z.reshape(nt, _TILE, d)
    z2 = jnp.sum(z * z, axis=-1).reshape(nt, 1, _TILE)
    m3 = mask.astype(jnp.float32).reshape(nt, 1, _TILE)
    c2 = jnp.sum(codebook * codebook, axis=-1).reshape(k, 1)
    kiota = jax.lax.broadcasted_iota(jnp.int32, (k, 1), 0)

    idx3, sumsq, cnt = pl.pallas_call(
        _vq_step,
        grid=(nt,),
        in_specs=[
            pl.BlockSpec((1, _TILE, d), lambda i: (i, 0, 0)),
            pl.BlockSpec((1, 1, _TILE), lambda i: (i, 0, 0)),
            pl.BlockSpec((1, 1, _TILE), lambda i: (i, 0, 0)),
            pl.BlockSpec((k, d), lambda i: (0, 0)),
            pl.BlockSpec((k, 1), lambda i: (0, 0)),
            pl.BlockSpec((k, 1), lambda i: (0, 0)),
        ],
        out_specs=[
            pl.BlockSpec((1, 1, _TILE), lambda i: (i, 0, 0)),
            pl.BlockSpec((1, 1), lambda i: (0, 0)),
            pl.BlockSpec((1, 1), lambda i: (0, 0)),
        ],
        out_shape=[
            jax.ShapeDtypeStruct((nt, 1, _TILE), jnp.int32),
            jax.ShapeDtypeStruct((1, 1), jnp.float32),
            jax.ShapeDtypeStruct((1, 1), jnp.float32),
        ],
        compiler_params=pltpu.CompilerParams(
            dimension_semantics=("arbitrary",),
        ),
    )(z3, z2, m3, codebook, c2, kiota)
    return idx3.reshape(b, t), sumsq[0, 0], cnt[0, 0]


def _sc_gather(codebook, indices):
    nrows, d = codebook.shape[0], codebook.shape[1]
    nidx = indices.shape[0]
    info = plsc.get_sparse_core_info()
    nw = info.num_cores * info.num_subcores
    b_per_w = nidx // nw
    nchunks = b_per_w // _CHUNK
    mesh = plsc.VectorSubcoreMesh(core_axis_name="c", subcore_axis_name="s")

    @functools.partial(
        pl.kernel, mesh=mesh,
        out_type=jax.ShapeDtypeStruct((nidx, d), jnp.float32),
        scratch_types=[
            pltpu.VMEM((_CHUNK,), jnp.int32),
            pltpu.VMEM((_CHUNK,), jnp.int32),
            pltpu.VMEM((_CHUNK, d), jnp.float32),
            pltpu.VMEM((_CHUNK, d), jnp.float32),
            pltpu.SemaphoreType.DMA,
            pltpu.SemaphoreType.DMA,
        ],
    )
    def gather_k(cb_hbm, idx_hbm, out_hbm,
                 idx0, idx1, rows0, rows1, sem0, sem1):
        wid = lax.axis_index("s") * info.num_cores + lax.axis_index("c")
        base = wid * b_per_w
        bufs = ((idx0, rows0, sem0), (idx1, rows1, sem1))
        pending = []
        for j in range(nchunks):
            iv, rv, sem = bufs[j % 2]
            off = base + j * _CHUNK
            pltpu.sync_copy(idx_hbm.at[pl.ds(off, _CHUNK)], iv)
            cp = pltpu.async_copy(cb_hbm.at[iv], rv, sem)
            pending.append((cp, rv, off))
            if len(pending) == 2:
                pcp, prv, poff = pending.pop(0)
                pcp.wait()
                pltpu.sync_copy(prv, out_hbm.at[pl.ds(poff, _CHUNK)])
        for pcp, prv, poff in pending:
            pcp.wait()
            pltpu.sync_copy(prv, out_hbm.at[pl.ds(poff, _CHUNK)])

    return gather_k(codebook, indices)


def kernel(z, mask, codebook):
    b, t, d = z.shape
    indices, sumsq, cnt = _tc_indices_loss(z, mask, codebook)
    rows = _sc_gather(codebook, indices.reshape(b * t))
    quantized = rows.reshape(b, t, d)
    denom = jnp.maximum(cnt, 1.0) * jnp.float32(d)
    sum_commit_loss = sumsq / denom
    return quantized, indices, sum_commit_loss


# speedup vs baseline: 1.2080x; 1.0170x over previous
import functools

import jax
import jax.numpy as jnp
from jax import lax
from jax.experimental import pallas as pl
from jax.experimental.pallas import tpu as pltpu
from jax.experimental.pallas import tpu_sc as plsc

_TILE = 1024
_CHUNK = 128


def _vq_step(z_ref, z2_ref, m_ref, cb_ref, c2_ref, kiota_ref, w_ref,
             idx_ref, sumsq_ref, cnt_ref):
    i = pl.program_id(0)
    z = z_ref[0]
    k = cb_ref.shape[0]

    dots = jax.lax.dot_general(
        cb_ref[...], z, (((1,), (1,)), ((), ())),
        preferred_element_type=jnp.float32)
    dist = (z2_ref[0] - 2.0 * dots) + c2_ref[...]

    minv = jnp.min(dist, axis=0, keepdims=True)
    hits = (dist == minv).astype(jnp.bfloat16)
    stats = jax.lax.dot_general(
        w_ref[...], hits, (((1,), (0,)), ((), ())),
        preferred_element_type=jnp.float32)
    count = stats[0:1]
    idx_fast = stats[1:2] * 4.0 + stats[2:3]
    idx_ref[0] = idx_fast.astype(jnp.int32)

    @pl.when(jnp.max(count) > 1.0)
    def _tie_fallback():
        kiota = jnp.broadcast_to(kiota_ref[...], dist.shape)
        idx = jnp.min(jnp.where(dist == minv, kiota, k),
                      axis=0, keepdims=True)
        idx_ref[0] = idx

    m = m_ref[0]
    psum = jnp.sum(minv * m, keepdims=True)
    pcnt = jnp.sum(m, keepdims=True)

    @pl.when(i == 0)
    def _init():
        sumsq_ref[...] = jnp.zeros((1, 1), jnp.float32)
        cnt_ref[...] = jnp.zeros((1, 1), jnp.float32)

    sumsq_ref[...] += psum
    cnt_ref[...] += pcnt


def _tc_indices_loss(z, mask, codebook):
    b, t, d = z.shape
    k = codebook.shape[0]
    n = b * t
    nt = n // _TILE

    z3 = z.reshape(nt, _TILE, d)
    z2 = jnp.sum(z * z, axis=-1).reshape(nt, 1, _TILE)
    m3 = mask.astype(jnp.float32).reshape(nt, 1, _TILE)
    c2 = jnp.sum(codebook * codebook, axis=-1).reshape(k, 1)
    kiota = jax.lax.broadcasted_iota(jnp.int32, (k, 1), 0)
    karr = jax.lax.iota(jnp.float32, k)
    w = jnp.zeros((8, k), jnp.float32)
    w = w.at[0].set(1.0).at[1].set(jnp.floor(karr / 4.0)).at[2].set(
        karr - 4.0 * jnp.floor(karr / 4.0))

    idx3, sumsq, cnt = pl.pallas_call(
        _vq_step,
        grid=(nt,),
        in_specs=[
            pl.BlockSpec((1, _TILE, d), lambda i: (i, 0, 0)),
            pl.BlockSpec((1, 1, _TILE), lambda i: (i, 0, 0)),
            pl.BlockSpec((1, 1, _TILE), lambda i: (i, 0, 0)),
            pl.BlockSpec((k, d), lambda i: (0, 0)),
            pl.BlockSpec((k, 1), lambda i: (0, 0)),
            pl.BlockSpec((k, 1), lambda i: (0, 0)),
            pl.BlockSpec((8, k), lambda i: (0, 0)),
        ],
        out_specs=[
            pl.BlockSpec((1, 1, _TILE), lambda i: (i, 0, 0)),
            pl.BlockSpec((1, 1), lambda i: (0, 0)),
            pl.BlockSpec((1, 1), lambda i: (0, 0)),
        ],
        out_shape=[
            jax.ShapeDtypeStruct((nt, 1, _TILE), jnp.int32),
            jax.ShapeDtypeStruct((1, 1), jnp.float32),
            jax.ShapeDtypeStruct((1, 1), jnp.float32),
        ],
        compiler_params=pltpu.CompilerParams(
            dimension_semantics=("arbitrary",),
        ),
    )(z3, z2, m3, codebook, c2, kiota, w)
    return idx3.reshape(b, t), sumsq[0, 0], cnt[0, 0]


def _sc_gather(codebook, indices):
    nrows, d = codebook.shape[0], codebook.shape[1]
    nidx = indices.shape[0]
    info = plsc.get_sparse_core_info()
    nw = info.num_cores * info.num_subcores
    b_per_w = nidx // nw
    nchunks = b_per_w // _CHUNK
    mesh = plsc.VectorSubcoreMesh(core_axis_name="c", subcore_axis_name="s")

    @functools.partial(
        pl.kernel, mesh=mesh,
        out_type=jax.ShapeDtypeStruct((nidx, d), jnp.float32),
        scratch_types=[
            pltpu.VMEM((_CHUNK,), jnp.int32),
            pltpu.VMEM((_CHUNK,), jnp.int32),
            pltpu.VMEM((_CHUNK, d), jnp.float32),
            pltpu.VMEM((_CHUNK, d), jnp.float32),
            pltpu.SemaphoreType.DMA,
            pltpu.SemaphoreType.DMA,
        ],
    )
    def gather_k(cb_hbm, idx_hbm, out_hbm,
                 idx0, idx1, rows0, rows1, sem0, sem1):
        wid = lax.axis_index("s") * info.num_cores + lax.axis_index("c")
        base = wid * b_per_w
        bufs = ((idx0, rows0, sem0), (idx1, rows1, sem1))
        pending = []
        for j in range(nchunks):
            iv, rv, sem = bufs[j % 2]
            off = base + j * _CHUNK
            pltpu.sync_copy(idx_hbm.at[pl.ds(off, _CHUNK)], iv)
            cp = pltpu.async_copy(cb_hbm.at[iv], rv, sem)
            pending.append((cp, rv, off))
            if len(pending) == 2:
                pcp, prv, poff = pending.pop(0)
                pcp.wait()
                pltpu.sync_copy(prv, out_hbm.at[pl.ds(poff, _CHUNK)])
        for pcp, prv, poff in pending:
            pcp.wait()
            pltpu.sync_copy(prv, out_hbm.at[pl.ds(poff, _CHUNK)])

    return gather_k(codebook, indices)


def kernel(z, mask, codebook):
    b, t, d = z.shape
    indices, sumsq, cnt = _tc_indices_loss(z, mask, codebook)
    rows = _sc_gather(codebook, indices.reshape(b * t))
    quantized = rows.reshape(b, t, d)
    denom = jnp.maximum(cnt, 1.0) * jnp.float32(d)
    sum_commit_loss = sumsq / denom
    return quantized, indices, sum_commit_loss


# speedup vs baseline: 1.2642x; 1.0466x over previous
import functools

import jax
import jax.numpy as jnp
from jax import lax
from jax.experimental import pallas as pl
from jax.experimental.pallas import tpu as pltpu
from jax.experimental.pallas import tpu_sc as plsc

_TILE = 2048
_CHUNK = 128


def _vq_step(z_ref, z2_ref, m_ref, cb_ref, c2_ref, kiota_ref, w_ref,
             idx_ref, sumsq_ref, cnt_ref):
    i = pl.program_id(0)
    z = z_ref[0]
    k = cb_ref.shape[0]

    dots = jax.lax.dot_general(
        cb_ref[...], z, (((1,), (1,)), ((), ())),
        preferred_element_type=jnp.float32)
    dist = (z2_ref[0] - 2.0 * dots) + c2_ref[...]

    minv = jnp.min(dist, axis=0, keepdims=True)
    hits = (dist == minv).astype(jnp.bfloat16)
    stats = jax.lax.dot_general(
        w_ref[...], hits, (((1,), (0,)), ((), ())),
        preferred_element_type=jnp.float32)
    count = stats[0:1]
    idx_fast = stats[1:2] * 4.0 + stats[2:3]
    idx_ref[0] = idx_fast.astype(jnp.int32)

    @pl.when(jnp.max(count) > 1.0)
    def _tie_fallback():
        kiota = jnp.broadcast_to(kiota_ref[...], dist.shape)
        idx = jnp.min(jnp.where(dist == minv, kiota, k),
                      axis=0, keepdims=True)
        idx_ref[0] = idx

    m = m_ref[0]
    psum = jnp.sum(minv * m, keepdims=True)
    pcnt = jnp.sum(m, keepdims=True)

    @pl.when(i == 0)
    def _init():
        sumsq_ref[...] = jnp.zeros((1, 1), jnp.float32)
        cnt_ref[...] = jnp.zeros((1, 1), jnp.float32)

    sumsq_ref[...] += psum
    cnt_ref[...] += pcnt


def _tc_indices_loss(z, mask, codebook):
    b, t, d = z.shape
    k = codebook.shape[0]
    n = b * t
    nt = n // _TILE

    z3 = z.reshape(nt, _TILE, d)
    z2 = jnp.sum(z * z, axis=-1).reshape(nt, 1, _TILE)
    m3 = mask.astype(jnp.float32).reshape(nt, 1, _TILE)
    c2 = jnp.sum(codebook * codebook, axis=-1).reshape(k, 1)
    kiota = jax.lax.broadcasted_iota(jnp.int32, (k, 1), 0)
    karr = jax.lax.iota(jnp.float32, k)
    w = jnp.zeros((8, k), jnp.float32)
    w = w.at[0].set(1.0).at[1].set(jnp.floor(karr / 4.0)).at[2].set(
        karr - 4.0 * jnp.floor(karr / 4.0))

    idx3, sumsq, cnt = pl.pallas_call(
        _vq_step,
        grid=(nt,),
        in_specs=[
            pl.BlockSpec((1, _TILE, d), lambda i: (i, 0, 0)),
            pl.BlockSpec((1, 1, _TILE), lambda i: (i, 0, 0)),
            pl.BlockSpec((1, 1, _TILE), lambda i: (i, 0, 0)),
            pl.BlockSpec((k, d), lambda i: (0, 0)),
            pl.BlockSpec((k, 1), lambda i: (0, 0)),
            pl.BlockSpec((k, 1), lambda i: (0, 0)),
            pl.BlockSpec((8, k), lambda i: (0, 0)),
        ],
        out_specs=[
            pl.BlockSpec((1, 1, _TILE), lambda i: (i, 0, 0)),
            pl.BlockSpec((1, 1), lambda i: (0, 0)),
            pl.BlockSpec((1, 1), lambda i: (0, 0)),
        ],
        out_shape=[
            jax.ShapeDtypeStruct((nt, 1, _TILE), jnp.int32),
            jax.ShapeDtypeStruct((1, 1), jnp.float32),
            jax.ShapeDtypeStruct((1, 1), jnp.float32),
        ],
        compiler_params=pltpu.CompilerParams(
            dimension_semantics=("arbitrary",),
        ),
    )(z3, z2, m3, codebook, c2, kiota, w)
    return idx3.reshape(b, t), sumsq[0, 0], cnt[0, 0]


def _sc_gather(codebook, indices):
    nrows, d = codebook.shape[0], codebook.shape[1]
    nidx = indices.shape[0]
    info = plsc.get_sparse_core_info()
    nw = info.num_cores * info.num_subcores
    b_per_w = nidx // nw
    nchunks = b_per_w // _CHUNK
    mesh = plsc.VectorSubcoreMesh(core_axis_name="c", subcore_axis_name="s")

    @functools.partial(
        pl.kernel, mesh=mesh,
        out_type=jax.ShapeDtypeStruct((nidx, d), jnp.float32),
        scratch_types=[
            pltpu.VMEM((b_per_w,), jnp.int32),
            pltpu.VMEM((_CHUNK, d), jnp.float32),
            pltpu.VMEM((_CHUNK, d), jnp.float32),
            pltpu.SemaphoreType.DMA,
            pltpu.SemaphoreType.DMA,
            pltpu.SemaphoreType.DMA,
            pltpu.SemaphoreType.DMA,
        ],
    )
    def gather_k(cb_hbm, idx_hbm, out_hbm,
                 idx_v, rows0, rows1, g0, g1, o0, o1):
        wid = lax.axis_index("s") * info.num_cores + lax.axis_index("c")
        base = wid * b_per_w
        pltpu.sync_copy(idx_hbm.at[pl.ds(base, b_per_w)], idx_v)
        bufs = ((rows0, g0, o0), (rows1, g1, o1))
        gathers = [None, None]
        writes = [None, None]
        for j in range(nchunks):
            s = j % 2
            rv, gs, os_ = bufs[s]
            if writes[s] is not None:
                writes[s].wait()
            gathers[s] = pltpu.async_copy(
                cb_hbm.at[idx_v.at[pl.ds(j * _CHUNK, _CHUNK)]], rv, gs)
            if j > 0:
                sp = (j - 1) % 2
                prv, _, pos = bufs[sp]
                gathers[sp].wait()
                writes[sp] = pltpu.async_copy(
                    prv, out_hbm.at[pl.ds(base + (j - 1) * _CHUNK, _CHUNK)],
                    pos)
        sl = (nchunks - 1) % 2
        lrv, _, los = bufs[sl]
        gathers[sl].wait()
        writes[sl] = pltpu.async_copy(
            lrv, out_hbm.at[pl.ds(base + (nchunks - 1) * _CHUNK, _CHUNK)],
            los)
        for w in writes:
            if w is not None:
                w.wait()

    return gather_k(codebook, indices)


def kernel(z, mask, codebook):
    b, t, d = z.shape
    indices, sumsq, cnt = _tc_indices_loss(z, mask, codebook)
    rows = _sc_gather(codebook, indices.reshape(b * t))
    quantized = rows.reshape(b, t, d)
    denom = jnp.maximum(cnt, 1.0) * jnp.float32(d)
    sum_commit_loss = sumsq / denom
    return quantized, indices, sum_commit_loss


# speedup vs baseline: 1.3387x; 1.0589x over previous
import functools

import jax
import jax.numpy as jnp
from jax import lax
from jax.experimental import pallas as pl
from jax.experimental.pallas import tpu as pltpu
from jax.experimental.pallas import tpu_sc as plsc

_TILE = 2048
_CHUNK = 128


def _vq_step(z_ref, m_ref, cb_ref, c2_ref, kiota_ref, w_ref,
             idx_ref, sumsq_ref, cnt_ref):
    i = pl.program_id(0)
    z = z_ref[0]
    k = cb_ref.shape[0]

    z2 = jnp.swapaxes(jnp.sum(z * z, axis=1, keepdims=True), 0, 1)
    dots = jax.lax.dot_general(
        cb_ref[...], z, (((1,), (1,)), ((), ())),
        preferred_element_type=jnp.float32)
    dist = (z2 - 2.0 * dots) + c2_ref[...]

    minv = jnp.min(dist, axis=0, keepdims=True)
    hits = (dist == minv).astype(jnp.bfloat16)
    stats = jax.lax.dot_general(
        w_ref[...], hits, (((1,), (0,)), ((), ())),
        preferred_element_type=jnp.float32)
    count = stats[0:1]
    idx_fast = stats[1:2] * 4.0 + stats[2:3]
    idx_ref[0] = idx_fast.astype(jnp.int32)

    @pl.when(jnp.max(count) > 1.0)
    def _tie_fallback():
        kiota = jnp.broadcast_to(kiota_ref[...], dist.shape)
        idx = jnp.min(jnp.where(dist == minv, kiota, k),
                      axis=0, keepdims=True)
        idx_ref[0] = idx

    m = m_ref[0]
    psum = jnp.sum(minv * m, keepdims=True)
    pcnt = jnp.sum(m, keepdims=True)

    @pl.when(i == 0)
    def _init():
        sumsq_ref[...] = jnp.zeros((1, 1), jnp.float32)
        cnt_ref[...] = jnp.zeros((1, 1), jnp.float32)

    sumsq_ref[...] += psum
    cnt_ref[...] += pcnt


def _tc_indices_loss(z, mask, codebook):
    b, t, d = z.shape
    k = codebook.shape[0]
    n = b * t
    nt = n // _TILE

    z3 = z.reshape(nt, _TILE, d)
    m3 = mask.astype(jnp.float32).reshape(nt, 1, _TILE)
    c2 = jnp.sum(codebook * codebook, axis=-1).reshape(k, 1)
    kiota = jax.lax.broadcasted_iota(jnp.int32, (k, 1), 0)
    karr = jax.lax.iota(jnp.float32, k)
    w = jnp.zeros((8, k), jnp.float32)
    w = w.at[0].set(1.0).at[1].set(jnp.floor(karr / 4.0)).at[2].set(
        karr - 4.0 * jnp.floor(karr / 4.0))

    idx3, sumsq, cnt = pl.pallas_call(
        _vq_step,
        grid=(nt,),
        in_specs=[
            pl.BlockSpec((1, _TILE, d), lambda i: (i, 0, 0)),
            pl.BlockSpec((1, 1, _TILE), lambda i: (i, 0, 0)),
            pl.BlockSpec((k, d), lambda i: (0, 0)),
            pl.BlockSpec((k, 1), lambda i: (0, 0)),
            pl.BlockSpec((k, 1), lambda i: (0, 0)),
            pl.BlockSpec((8, k), lambda i: (0, 0)),
        ],
        out_specs=[
            pl.BlockSpec((1, 1, _TILE), lambda i: (i, 0, 0)),
            pl.BlockSpec((1, 1), lambda i: (0, 0)),
            pl.BlockSpec((1, 1), lambda i: (0, 0)),
        ],
        out_shape=[
            jax.ShapeDtypeStruct((nt, 1, _TILE), jnp.int32),
            jax.ShapeDtypeStruct((1, 1), jnp.float32),
            jax.ShapeDtypeStruct((1, 1), jnp.float32),
        ],
        compiler_params=pltpu.CompilerParams(
            dimension_semantics=("arbitrary",),
        ),
    )(z3, m3, codebook, c2, kiota, w)
    return idx3.reshape(b, t), sumsq[0, 0], cnt[0, 0]


def _sc_gather(codebook, indices):
    nrows, d = codebook.shape[0], codebook.shape[1]
    nidx = indices.shape[0]
    info = plsc.get_sparse_core_info()
    nw = info.num_cores * info.num_subcores
    b_per_w = nidx // nw
    nchunks = b_per_w // _CHUNK
    mesh = plsc.VectorSubcoreMesh(core_axis_name="c", subcore_axis_name="s")

    @functools.partial(
        pl.kernel, mesh=mesh,
        out_type=jax.ShapeDtypeStruct((nidx, d), jnp.float32),
        scratch_types=[
            pltpu.VMEM((b_per_w,), jnp.int32),
            pltpu.VMEM((_CHUNK, d), jnp.float32),
            pltpu.VMEM((_CHUNK, d), jnp.float32),
            pltpu.SemaphoreType.DMA,
            pltpu.SemaphoreType.DMA,
            pltpu.SemaphoreType.DMA,
            pltpu.SemaphoreType.DMA,
        ],
    )
    def gather_k(cb_hbm, idx_hbm, out_hbm,
                 idx_v, rows0, rows1, g0, g1, o0, o1):
        wid = lax.axis_index("s") * info.num_cores + lax.axis_index("c")
        base = wid * b_per_w
        pltpu.sync_copy(idx_hbm.at[pl.ds(base, b_per_w)], idx_v)
        bufs = ((rows0, g0, o0), (rows1, g1, o1))
        gathers = [None, None]
        writes = [None, None]
        for j in range(nchunks):
            s = j % 2
            rv, gs, os_ = bufs[s]
            if writes[s] is not None:
                writes[s].wait()
            gathers[s] = pltpu.async_copy(
                cb_hbm.at[idx_v.at[pl.ds(j * _CHUNK, _CHUNK)]], rv, gs)
            if j > 0:
                sp = (j - 1) % 2
                prv, _, pos = bufs[sp]
                gathers[sp].wait()
                writes[sp] = pltpu.async_copy(
                    prv, out_hbm.at[pl.ds(base + (j - 1) * _CHUNK, _CHUNK)],
                    pos)
        sl = (nchunks - 1) % 2
        lrv, _, los = bufs[sl]
        gathers[sl].wait()
        writes[sl] = pltpu.async_copy(
            lrv, out_hbm.at[pl.ds(base + (nchunks - 1) * _CHUNK, _CHUNK)],
            los)
        for w in writes:
            if w is not None:
                w.wait()

    return gather_k(codebook, indices)


def kernel(z, mask, codebook):
    b, t, d = z.shape
    indices, sumsq, cnt = _tc_indices_loss(z, mask, codebook)
    rows = _sc_gather(codebook, indices.reshape(b * t))
    quantized = rows.reshape(b, t, d)
    denom = jnp.maximum(cnt, 1.0) * jnp.float32(d)
    sum_commit_loss = sumsq / denom
    return quantized, indices, sum_commit_loss


# speedup vs baseline: 1.3504x; 1.0088x over previous
import functools

import jax
import jax.numpy as jnp
from jax import lax
from jax.experimental import pallas as pl
from jax.experimental.pallas import tpu as pltpu
from jax.experimental.pallas import tpu_sc as plsc

_TILE = 2048
_CHUNK = 128


def _vq_step(z_ref, m_ref, cb_ref, c2_ref, kiota_ref, w_ref,
             idx_ref, sumsq_ref, cnt_ref):
    i = pl.program_id(0)
    z = z_ref[0]
    k = cb_ref.shape[0]

    z2 = jnp.swapaxes(jnp.sum(z * z, axis=1, keepdims=True), 0, 1)
    dots = jax.lax.dot_general(
        cb_ref[...], z, (((1,), (1,)), ((), ())),
        preferred_element_type=jnp.float32)
    dist = (z2 - 2.0 * dots) + c2_ref[...]

    minv = jnp.min(dist, axis=0, keepdims=True)
    hits = (dist == minv).astype(jnp.bfloat16)
    stats = jax.lax.dot_general(
        w_ref[...], hits, (((1,), (0,)), ((), ())),
        preferred_element_type=jnp.float32)
    count = stats[0:1]
    idx_fast = stats[1:2] * 4.0 + stats[2:3]
    idx_ref[0] = idx_fast.astype(jnp.int32)

    @pl.when(jnp.max(count) > 1.0)
    def _tie_fallback():
        kiota = jnp.broadcast_to(kiota_ref[...], dist.shape)
        idx = jnp.min(jnp.where(dist == minv, kiota, k),
                      axis=0, keepdims=True)
        idx_ref[0] = idx

    m = m_ref[0]
    psum = jnp.sum(minv * m, keepdims=True)
    pcnt = jnp.sum(m, keepdims=True)

    @pl.when(i == 0)
    def _init():
        sumsq_ref[...] = jnp.zeros((1, 1), jnp.float32)
        cnt_ref[...] = jnp.zeros((1, 1), jnp.float32)

    sumsq_ref[...] += psum
    cnt_ref[...] += pcnt


def _tc_indices_loss(z, mask, codebook):
    b, t, d = z.shape
    k = codebook.shape[0]
    n = b * t
    nt = n // _TILE

    z3 = z.reshape(nt, _TILE, d)
    m3 = mask.astype(jnp.float32).reshape(nt, 1, _TILE)
    c2 = jnp.sum(codebook * codebook, axis=-1).reshape(k, 1)
    kiota = jax.lax.broadcasted_iota(jnp.int32, (k, 1), 0)
    karr = jax.lax.iota(jnp.float32, k)
    w = jnp.zeros((8, k), jnp.float32)
    w = w.at[0].set(1.0).at[1].set(jnp.floor(karr / 4.0)).at[2].set(
        karr - 4.0 * jnp.floor(karr / 4.0))

    idx3, sumsq, cnt = pl.pallas_call(
        _vq_step,
        grid=(nt,),
        in_specs=[
            pl.BlockSpec((1, _TILE, d), lambda i: (i, 0, 0)),
            pl.BlockSpec((1, 1, _TILE), lambda i: (i, 0, 0)),
            pl.BlockSpec((k, d), lambda i: (0, 0)),
            pl.BlockSpec((k, 1), lambda i: (0, 0)),
            pl.BlockSpec((k, 1), lambda i: (0, 0)),
            pl.BlockSpec((8, k), lambda i: (0, 0)),
        ],
        out_specs=[
            pl.BlockSpec((1, 1, _TILE), lambda i: (i, 0, 0)),
            pl.BlockSpec((1, 1), lambda i: (0, 0)),
            pl.BlockSpec((1, 1), lambda i: (0, 0)),
        ],
        out_shape=[
            jax.ShapeDtypeStruct((nt, 1, _TILE), jnp.int32),
            jax.ShapeDtypeStruct((1, 1), jnp.float32),
            jax.ShapeDtypeStruct((1, 1), jnp.float32),
        ],
        compiler_params=pltpu.CompilerParams(
            dimension_semantics=("arbitrary",),
        ),
    )(z3, m3, codebook, c2, kiota, w)
    return idx3.reshape(b, t), sumsq[0, 0], cnt[0, 0]


def _sc_gather(codebook, indices):
    nrows, d = codebook.shape[0], codebook.shape[1]
    nidx = indices.shape[0]
    info = plsc.get_sparse_core_info()
    nw = info.num_cores * info.num_subcores
    b_per_w = nidx // nw
    nchunks = b_per_w // _CHUNK
    mesh = plsc.VectorSubcoreMesh(core_axis_name="c", subcore_axis_name="s")

    @functools.partial(
        pl.kernel, mesh=mesh,
        out_type=jax.ShapeDtypeStruct((nidx, d), jnp.float32),
        scratch_types=[
            pltpu.VMEM((b_per_w,), jnp.int32),
            pltpu.VMEM((_CHUNK, d), jnp.float32),
            pltpu.VMEM((_CHUNK, d), jnp.float32),
            pltpu.VMEM((_CHUNK, d), jnp.float32),
            pltpu.SemaphoreType.DMA,
            pltpu.SemaphoreType.DMA,
            pltpu.SemaphoreType.DMA,
            pltpu.SemaphoreType.DMA,
            pltpu.SemaphoreType.DMA,
            pltpu.SemaphoreType.DMA,
        ],
    )
    def gather_k(cb_hbm, idx_hbm, out_hbm,
                 idx_v, rows0, rows1, rows2, g0, g1, g2, o0, o1, o2):
        wid = lax.axis_index("s") * info.num_cores + lax.axis_index("c")
        base = wid * b_per_w
        pltpu.sync_copy(idx_hbm.at[pl.ds(base, b_per_w)], idx_v)
        bufs = ((rows0, g0, o0), (rows1, g1, o1), (rows2, g2, o2))
        gathers = [None, None, None]
        writes = [None, None, None]

        def flush(jj):
            sp = jj % 3
            prv, _, pos = bufs[sp]
            gathers[sp].wait()
            writes[sp] = pltpu.async_copy(
                prv, out_hbm.at[pl.ds(base + jj * _CHUNK, _CHUNK)], pos)

        for j in range(nchunks):
            s = j % 3
            rv, gs, _ = bufs[s]
            if writes[s] is not None:
                writes[s].wait()
            gathers[s] = pltpu.async_copy(
                cb_hbm.at[idx_v.at[pl.ds(j * _CHUNK, _CHUNK)]], rv, gs)
            if j >= 2:
                flush(j - 2)
        for jj in range(max(nchunks - 2, 0), nchunks):
            flush(jj)
        for w in writes:
            if w is not None:
                w.wait()

    return gather_k(codebook, indices)


def kernel(z, mask, codebook):
    b, t, d = z.shape
    indices, sumsq, cnt = _tc_indices_loss(z, mask, codebook)
    rows = _sc_gather(codebook, indices.reshape(b * t))
    quantized = rows.reshape(b, t, d)
    denom = jnp.maximum(cnt, 1.0) * jnp.float32(d)
    sum_commit_loss = sumsq / denom
    return quantized, indices, sum_commit_loss
